# baseline (speedup 1.0000x reference)
import jax
import jax.numpy as jnp
from jax import lax
from jax.experimental import pallas as pl

_A = 5
_H = 26
_W = 26
_T = 30
_HW = _H * _W
_POS_THRESH = 0.6


def _body(xy_ref, wh_ref, obj_ref, truth_ref, bias_ref,
          txy_ref, twh_ref, tw_ref, tobj_ref, tnoobj_ref, tlabel_ref):
    f32 = jnp.float32
    xf = xy_ref[0].reshape(2 * _A, _HW)
    whf = wh_ref[0].reshape(2 * _A, _HW)
    objf = obj_ref[0].reshape(_A, _HW)

    tx = truth_ref[0, :, 0:1]
    ty = truth_ref[0, :, 1:2]
    tw = truth_ref[0, :, 2:3]
    th = truth_ref[0, :, 3:4]
    tcls = truth_ref[0, :, 4:5]
    valid = tw > 1e-6

    twc = tw * _W
    thc = th * _H
    ci = jnp.clip((tx * _W).astype(jnp.int32), 0, _W - 1)
    cj = jnp.clip((ty * _H).astype(jnp.int32), 0, _H - 1)
    tgt_x = tx * _W - ci.astype(f32)
    tgt_y = ty * _H - cj.astype(f32)
    wgt = 2.0 - tw * th

    best_r = jnp.full_like(tx, -1.0)
    ba = jnp.zeros_like(ci)
    bw_sel = jnp.zeros_like(tx)
    bh_sel = jnp.zeros_like(tx)
    for a in range(_A):
        bw_a = bias_ref[0:1, 2 * a:2 * a + 1]
        bh_a = bias_ref[0:1, 2 * a + 1:2 * a + 2]
        inter = jnp.minimum(twc, bw_a) * jnp.minimum(thc, bh_a)
        union = twc * thc + bw_a * bh_a - inter
        rr = inter / jnp.maximum(union, 1e-12)
        upd = rr > best_r
        best_r = jnp.where(upd, rr, best_r)
        ba = jnp.where(upd, a, ba)
        bw_sel = jnp.where(upd, bw_a, bw_sel)
        bh_sel = jnp.where(upd, bh_a, bh_sel)
    tgt_w = jnp.log(jnp.maximum(twc, 1e-12) / bw_sel)
    tgt_h = jnp.log(jnp.maximum(thc, 1e-12) / bh_sel)

    m = jnp.where(valid, ba * _HW + cj * _W + ci, -1)
    mT = jnp.transpose(m, (1, 0))
    tcol = lax.broadcasted_iota(jnp.int32, (_T, 1), 0)
    tcolT = jnp.transpose(tcol, (1, 0))
    dup = jnp.max(((m == mT) & (tcolT > tcol)).astype(jnp.int32),
                  axis=1, keepdims=True)
    wins = dup == 0

    half_tw = tw * 0.5
    half_th = th * 0.5
    tl = jnp.where(valid, tx - half_tw, 1e30)
    tr = jnp.where(valid, tx + half_tw, -1e30)
    tt = ty - half_th
    tb = ty + half_th
    t_area = tw * th
    c1 = t_area * (_POS_THRESH / (1.0 + _POS_THRESH))

    ones_col = jnp.ones((_T, 1), dtype=f32)
    vals = jnp.concatenate(
        [tgt_x, tgt_y, tgt_w, tgt_h, wgt, tcls, t_area, ones_col], axis=1)

    idx_row = lax.broadcasted_iota(jnp.int32, (1, _HW), 1)
    gx = (idx_row % _W).astype(f32)
    gy = (idx_row // _W).astype(f32)

    txy_rows = []
    twh_rows = []
    tw_rows = []
    tobj_rows = []
    tnoobj_rows = []
    tlabel_rows = []
    for a in range(_A):
        xy0 = xf[2 * a:2 * a + 1]
        xy1 = xf[2 * a + 1:2 * a + 2]
        wh0 = whf[2 * a:2 * a + 1]
        wh1 = whf[2 * a + 1:2 * a + 2]
        obj_a = objf[a:a + 1]
        bw_a = bias_ref[0:1, 2 * a:2 * a + 1]
        bh_a = bias_ref[0:1, 2 * a + 1:2 * a + 2]

        px = (gx + xy0) * (1.0 / _W)
        py = (gy + xy1) * (1.0 / _H)
        pw = jnp.exp(wh0) * (bw_a * (1.0 / _W))
        ph = jnp.exp(wh1) * (bh_a * (1.0 / _H))
        p_l = px - pw * 0.5
        p_r = px + pw * 0.5
        p_t = py - ph * 0.5
        p_b = py + ph * 0.5
        p_area = pw * ph
        c0 = p_area * (_POS_THRESH / (1.0 + _POS_THRESH))

        l = jnp.maximum(p_l, tl)
        r = jnp.minimum(p_r, tr)
        t = jnp.maximum(p_t, tt)
        b = jnp.minimum(p_b, tb)
        inter = jnp.maximum(r - l, 0.0) * jnp.maximum(b - t, 0.0)
        over_f = (inter > (c0 + c1)).astype(f32)
        any_over = jnp.max(over_f, axis=0, keepdims=True) > 0.0

        wf = ((m == idx_row + (a * _HW)) & wins).astype(f32)
        picked = lax.dot_general(vals, wf, (((0,), (0,)), ((), ())),
                                 preferred_element_type=f32,
                                 precision=lax.Precision.HIGHEST)
        inter_w = jnp.sum(wf * inter, axis=0, keepdims=True)

        assigned = picked[7:8] > 0.0
        union_w = (p_area + picked[6:7]) - inter_w
        iou_w = inter_w / jnp.maximum(union_w, 1e-12)

        txy_rows.append(jnp.where(assigned, picked[0:1], xy0))
        txy_rows.append(jnp.where(assigned, picked[1:2], xy1))
        twh_rows.append(jnp.where(assigned, picked[2:3], wh0))
        twh_rows.append(jnp.where(assigned, picked[3:4], wh1))
        g_row = jnp.where(assigned, picked[4:5], 0.0)
        tw_rows.append(g_row)
        tw_rows.append(g_row)
        tobj_rows.append(jnp.where(assigned, iou_w, obj_a))
        tnoobj_rows.append(jnp.where(assigned | any_over, obj_a, 0.0))
        tlabel_rows.append(jnp.where(assigned, picked[5:6], -1.0))

    txy_ref[0] = jnp.concatenate(txy_rows, axis=0).reshape(2 * _A, _H, _W)
    twh_ref[0] = jnp.concatenate(twh_rows, axis=0).reshape(2 * _A, _H, _W)
    tw_ref[0] = jnp.concatenate(tw_rows, axis=0).reshape(2 * _A, _H, _W)
    tobj_ref[0] = jnp.concatenate(tobj_rows, axis=0).reshape(_A, _H, _W)
    tnoobj_ref[0] = jnp.concatenate(tnoobj_rows, axis=0).reshape(_A, _H, _W)
    tlabel_ref[0] = jnp.concatenate(tlabel_rows, axis=0).reshape(_A, _H, _W)


def kernel(xy, wh, obj, truth, biases):
    B = xy.shape[0]
    f32 = jnp.float32
    bias_r = biases.reshape(1, 2 * _A)

    out_shapes = (
        jax.ShapeDtypeStruct((B, 2 * _A, _H, _W), f32),
        jax.ShapeDtypeStruct((B, 2 * _A, _H, _W), f32),
        jax.ShapeDtypeStruct((B, 2 * _A, _H, _W), f32),
        jax.ShapeDtypeStruct((B, _A, _H, _W), f32),
        jax.ShapeDtypeStruct((B, _A, _H, _W), f32),
        jax.ShapeDtypeStruct((B, _A, _H, _W), f32),
    )
    in_specs = [
        pl.BlockSpec((1, 2 * _A, _H, _W), lambda b: (b, 0, 0, 0)),
        pl.BlockSpec((1, 2 * _A, _H, _W), lambda b: (b, 0, 0, 0)),
        pl.BlockSpec((1, _A, _H, _W), lambda b: (b, 0, 0, 0)),
        pl.BlockSpec((1, _T, 5), lambda b: (b, 0, 0)),
        pl.BlockSpec((1, 2 * _A), lambda b: (0, 0)),
    ]
    out_specs = (
        pl.BlockSpec((1, 2 * _A, _H, _W), lambda b: (b, 0, 0, 0)),
        pl.BlockSpec((1, 2 * _A, _H, _W), lambda b: (b, 0, 0, 0)),
        pl.BlockSpec((1, 2 * _A, _H, _W), lambda b: (b, 0, 0, 0)),
        pl.BlockSpec((1, _A, _H, _W), lambda b: (b, 0, 0, 0)),
        pl.BlockSpec((1, _A, _H, _W), lambda b: (b, 0, 0, 0)),
        pl.BlockSpec((1, _A, _H, _W), lambda b: (b, 0, 0, 0)),
    )
    return pl.pallas_call(
        _body,
        grid=(B,),
        in_specs=in_specs,
        out_specs=out_specs,
        out_shape=out_shapes,
    )(xy, wh, obj, truth, bias_r)


# speedup vs baseline: 1.0375x; 1.0375x over previous
import functools

import jax
import jax.numpy as jnp
from jax import lax
from jax.experimental import pallas as pl
from jax.experimental.pallas import tpu as pltpu
from jax.experimental.pallas import tpu_sc as plsc

_A = 5
_H = 26
_W = 26
_T = 30
_HW = _H * _W
_C = _A * _HW
_POS_THRESH = 0.6
_NROW = 16


def _tc_body(xy_ref, wh_ref, obj_ref, truth_ref, bias_ref, brow_ref,
             tnoobj_ref, tab_ref):
    f32 = jnp.float32
    tx = truth_ref[0, :, 0:1]
    ty = truth_ref[0, :, 1:2]
    tw = truth_ref[0, :, 2:3]
    th = truth_ref[0, :, 3:4]
    tcls = truth_ref[0, :, 4:5]
    valid = tw > 1e-6

    twc = tw * _W
    thc = th * _H
    ci = jnp.clip((tx * _W).astype(jnp.int32), 0, _W - 1)
    cj = jnp.clip((ty * _H).astype(jnp.int32), 0, _H - 1)
    tgt_x = tx * _W - ci.astype(f32)
    tgt_y = ty * _H - cj.astype(f32)
    wgt = 2.0 - tw * th

    best_r = jnp.full_like(tx, -1.0)
    ba = jnp.zeros_like(ci)
    bw_sel = jnp.zeros_like(tx)
    bh_sel = jnp.zeros_like(tx)
    for a in range(_A):
        bw_a = bias_ref[0:1, 2 * a:2 * a + 1]
        bh_a = bias_ref[0:1, 2 * a + 1:2 * a + 2]
        inter = jnp.minimum(twc, bw_a) * jnp.minimum(thc, bh_a)
        union = twc * thc + bw_a * bh_a - inter
        rr = inter / jnp.maximum(union, 1e-12)
        upd = rr > best_r
        best_r = jnp.where(upd, rr, best_r)
        ba = jnp.where(upd, a, ba)
        bw_sel = jnp.where(upd, bw_a, bw_sel)
        bh_sel = jnp.where(upd, bh_a, bh_sel)
    tgt_w = jnp.log(jnp.maximum(twc, 1e-12) / bw_sel)
    tgt_h = jnp.log(jnp.maximum(thc, 1e-12) / bh_sel)

    m = jnp.where(valid, ba * _HW + cj * _W + ci, -1)
    mT = jnp.transpose(m, (1, 0))
    tcol = lax.broadcasted_iota(jnp.int32, (_T, 1), 0)
    tcolT = jnp.transpose(tcol, (1, 0))
    dup = jnp.max(((m == mT) & (tcolT > tcol)).astype(jnp.int32),
                  axis=1, keepdims=True)
    wins = dup == 0
    m_f = jnp.where(wins, m, -1).astype(f32)

    pad3 = jnp.full((_T, _NROW - 13), -1.0, dtype=f32)
    tab = jnp.concatenate(
        [m_f, tgt_x, tgt_y, tgt_w, tgt_h, wgt, tcls,
         tx, ty, tw, th, bw_sel, bh_sel, pad3], axis=1)
    tab = jnp.transpose(tab, (1, 0))
    tab = jnp.concatenate(
        [tab, jnp.full((_NROW, 2), -1.0, dtype=f32)], axis=1)
    tab_ref[0] = tab

    half_tw = tw * 0.5
    half_th = th * 0.5
    tl = jnp.where(valid, tx - half_tw, 1e30)
    tr = jnp.where(valid, tx + half_tw, -1e30)
    tt = ty - half_th
    tb = ty + half_th
    c1 = (tw * th) * (_POS_THRESH / (1.0 + _POS_THRESH))

    idx_row = lax.broadcasted_iota(jnp.int32, (1, _C), 1)
    gx = (idx_row % _W).astype(f32)
    gy = ((idx_row // _W) % _H).astype(f32)

    xy0 = xy_ref[0, 0:1, :]
    xy1 = xy_ref[0, 1:2, :]
    wh0 = wh_ref[0, 0:1, :]
    wh1 = wh_ref[0, 1:2, :]
    obj_r = obj_ref[0, 0:1, :]
    bw_row = brow_ref[0, 0:1, :]
    bh_row = brow_ref[0, 1:2, :]

    px = (gx + xy0) * (1.0 / _W)
    py = (gy + xy1) * (1.0 / _H)
    pw = jnp.exp(wh0) * bw_row
    ph = jnp.exp(wh1) * bh_row
    p_l = px - pw * 0.5
    p_r = px + pw * 0.5
    p_t = py - ph * 0.5
    p_b = py + ph * 0.5
    c0 = (pw * ph) * (_POS_THRESH / (1.0 + _POS_THRESH))

    l = jnp.maximum(p_l, tl)
    r = jnp.minimum(p_r, tr)
    t = jnp.maximum(p_t, tt)
    b = jnp.minimum(p_b, tb)
    inter = jnp.maximum(r - l, 0.0) * jnp.maximum(b - t, 0.0)
    over_f = (inter > (c0 + c1)).astype(f32)
    any_over = jnp.max(over_f, axis=0, keepdims=True) > 0.0
    tnoobj_ref[0, 0:1, :] = jnp.where(any_over, obj_r, 0.0)


def _sc_assign(B):
    f32 = jnp.float32
    mesh = plsc.VectorSubcoreMesh(core_axis_name="c", subcore_axis_name="s")
    n_tec = 32
    img_per_tec = B // n_tec
    out_t = tuple(jax.ShapeDtypeStruct(s, f32) for s in
                  [(2 * B, _C), (2 * B, _C), (B, _C), (B, _C), (B, _C),
                   (B, _C)])
    scr = ([pltpu.VMEM((_C,), f32) for _ in range(8)] +
           [pltpu.VMEM((_NROW, 2 * _T - 28), f32)])

    @functools.partial(
        pl.kernel, mesh=mesh, out_type=out_t, scratch_types=scr,
        compiler_params=pltpu.CompilerParams(needs_layout_passes=False))
    def k(xy2, wh2, obj1, noobj1, glc, tab3,
          o_xy, o_wh, o_g, o_obj, o_noobj, o_lab,
          s_x0, s_x1, s_w0, s_w1, s_obj, s_noobj, s_g, s_lab, s_tab):
        wid = lax.axis_index("s") * 2 + lax.axis_index("c")
        for j in range(img_per_tec):
            bimg = wid * img_per_tec + j
            pltpu.sync_copy(xy2.at[2 * bimg], s_x0)
            pltpu.sync_copy(xy2.at[2 * bimg + 1], s_x1)
            pltpu.sync_copy(wh2.at[2 * bimg], s_w0)
            pltpu.sync_copy(wh2.at[2 * bimg + 1], s_w1)
            pltpu.sync_copy(obj1.at[bimg], s_obj)
            pltpu.sync_copy(noobj1.at[bimg], s_noobj)
            pltpu.sync_copy(glc.at[0], s_g)
            pltpu.sync_copy(glc.at[1], s_lab)
            pltpu.sync_copy(tab3.at[bimg], s_tab)
            for v in range(2):
                sl = pl.ds(16 * v, 16)
                mi = s_tab[0, sl].astype(jnp.int32)
                mk = mi >= 0
                safe = jnp.maximum(mi, 0)
                tgx = s_tab[1, sl]
                tgy = s_tab[2, sl]
                tgw = s_tab[3, sl]
                tgh = s_tab[4, sl]
                wgt = s_tab[5, sl]
                tcl = s_tab[6, sl]
                txv = s_tab[7, sl]
                tyv = s_tab[8, sl]
                twv = s_tab[9, sl]
                thv = s_tab[10, sl]
                bwv = s_tab[11, sl]
                bhv = s_tab[12, sl]

                x0 = plsc.load_gather(s_x0, [safe])
                x1 = plsc.load_gather(s_x1, [safe])
                w0 = plsc.load_gather(s_w0, [safe])
                w1 = plsc.load_gather(s_w1, [safe])
                ov = plsc.load_gather(s_obj, [safe])

                gxv = (safe % 26).astype(f32)
                gyv = ((safe // 26) % 26).astype(f32)
                px = (gxv + x0) * (1.0 / _W)
                py = (gyv + x1) * (1.0 / _H)
                pw = jnp.exp(w0) * bwv * (1.0 / _W)
                ph = jnp.exp(w1) * bhv * (1.0 / _H)
                ll = jnp.maximum(px - pw * 0.5, txv - twv * 0.5)
                rr = jnp.minimum(px + pw * 0.5, txv + twv * 0.5)
                tt = jnp.maximum(py - ph * 0.5, tyv - thv * 0.5)
                bb = jnp.minimum(py + ph * 0.5, tyv + thv * 0.5)
                inter = (jnp.maximum(rr - ll, 0.0) *
                         jnp.maximum(bb - tt, 0.0))
                union = pw * ph + twv * thv - inter
                iou = jnp.where(union > 0,
                                inter / jnp.maximum(union, 1e-12), 0.0)

                plsc.store_scatter(s_x0, [safe], tgx, mask=mk)
                plsc.store_scatter(s_x1, [safe], tgy, mask=mk)
                plsc.store_scatter(s_w0, [safe], tgw, mask=mk)
                plsc.store_scatter(s_w1, [safe], tgh, mask=mk)
                plsc.store_scatter(s_g, [safe], wgt, mask=mk)
                plsc.store_scatter(s_obj, [safe], iou, mask=mk)
                plsc.store_scatter(s_noobj, [safe], ov, mask=mk)
                plsc.store_scatter(s_lab, [safe], tcl, mask=mk)

            pltpu.sync_copy(s_x0, o_xy.at[2 * bimg])
            pltpu.sync_copy(s_x1, o_xy.at[2 * bimg + 1])
            pltpu.sync_copy(s_w0, o_wh.at[2 * bimg])
            pltpu.sync_copy(s_w1, o_wh.at[2 * bimg + 1])
            pltpu.sync_copy(s_g, o_g.at[bimg])
            pltpu.sync_copy(s_obj, o_obj.at[bimg])
            pltpu.sync_copy(s_noobj, o_noobj.at[bimg])
            pltpu.sync_copy(s_lab, o_lab.at[bimg])

    return k


def kernel(xy, wh, obj, truth, biases):
    B = xy.shape[0]
    f32 = jnp.float32
    xy_t = xy.reshape(B, _A, 2, _HW).transpose(0, 2, 1, 3).reshape(B, 2, _C)
    wh_t = wh.reshape(B, _A, 2, _HW).transpose(0, 2, 1, 3).reshape(B, 2, _C)
    obj_r = obj.reshape(B, 1, _C)
    bias_r = biases.reshape(1, 2 * _A)
    bi = biases.reshape(_A, 2)
    brow = jnp.stack([jnp.repeat(bi[:, 0] * (1.0 / _W), _HW),
                      jnp.repeat(bi[:, 1] * (1.0 / _H), _HW)], axis=0)
    brow = brow.reshape(1, 2, _C)

    tnoobj_base, table = pl.pallas_call(
        _tc_body,
        grid=(B,),
        in_specs=[
            pl.BlockSpec((1, 2, _C), lambda b: (b, 0, 0)),
            pl.BlockSpec((1, 2, _C), lambda b: (b, 0, 0)),
            pl.BlockSpec((1, 1, _C), lambda b: (b, 0, 0)),
            pl.BlockSpec((1, _T, 5), lambda b: (b, 0, 0)),
            pl.BlockSpec((1, 2 * _A), lambda b: (0, 0)),
            pl.BlockSpec((1, 2, _C), lambda b: (0, 0, 0)),
        ],
        out_specs=(
            pl.BlockSpec((1, 1, _C), lambda b: (b, 0, 0)),
            pl.BlockSpec((1, _NROW, 2 * _T - 28), lambda b: (b, 0, 0)),
        ),
        out_shape=(
            jax.ShapeDtypeStruct((B, 1, _C), f32),
            jax.ShapeDtypeStruct((B, _NROW, 2 * _T - 28), f32),
        ),
    )(xy_t, wh_t, obj_r, truth, bias_r, brow)

    glc = jnp.stack([jnp.zeros((_C,), f32), jnp.full((_C,), -1.0, f32)])
    xyo, who, g, tobj, tnoobj, tlabel = _sc_assign(B)(
        xy_t.reshape(2 * B, _C), wh_t.reshape(2 * B, _C),
        obj_r.reshape(B, _C), tnoobj_base.reshape(B, _C), glc, table)

    t_xy = xyo.reshape(B, 2, _A, _HW).transpose(0, 2, 1, 3).reshape(
        B, 2 * _A, _H, _W)
    t_wh = who.reshape(B, 2, _A, _HW).transpose(0, 2, 1, 3).reshape(
        B, 2 * _A, _H, _W)
    g4 = g.reshape(B, 1, _A, _HW)
    t_w = jnp.broadcast_to(g4, (B, 2, _A, _HW)).transpose(0, 2, 1, 3).reshape(
        B, 2 * _A, _H, _W)
    return (
        t_xy,
        t_wh,
        t_w,
        tobj.reshape(B, _A, _H, _W),
        tnoobj.reshape(B, _A, _H, _W),
        tlabel.reshape(B, _A, _H, _W),
    )


# speedup vs baseline: 1.0898x; 1.0504x over previous
import functools

import jax
import jax.numpy as jnp
from jax import lax
from jax.experimental import pallas as pl
from jax.experimental.pallas import tpu as pltpu
from jax.experimental.pallas import tpu_sc as plsc

_A = 5
_H = 26
_W = 26
_T = 30
_HW = _H * _W
_C = _A * _HW
_POS_THRESH = 0.6
_NROW = 16


def _tc_body(xy_ref, wh_ref, obj_ref, truth_ref, bias_ref, brow_ref,
             tnoobj_ref, tab_ref):
    f32 = jnp.float32
    tx = truth_ref[0, :, 0:1]
    ty = truth_ref[0, :, 1:2]
    tw = truth_ref[0, :, 2:3]
    th = truth_ref[0, :, 3:4]
    tcls = truth_ref[0, :, 4:5]
    valid = tw > 1e-6

    twc = tw * _W
    thc = th * _H
    ci = jnp.clip((tx * _W).astype(jnp.int32), 0, _W - 1)
    cj = jnp.clip((ty * _H).astype(jnp.int32), 0, _H - 1)
    tgt_x = tx * _W - ci.astype(f32)
    tgt_y = ty * _H - cj.astype(f32)
    wgt = 2.0 - tw * th

    best_r = jnp.full_like(tx, -1.0)
    ba = jnp.zeros_like(ci)
    bw_sel = jnp.zeros_like(tx)
    bh_sel = jnp.zeros_like(tx)
    for a in range(_A):
        bw_a = bias_ref[0:1, 2 * a:2 * a + 1]
        bh_a = bias_ref[0:1, 2 * a + 1:2 * a + 2]
        inter = jnp.minimum(twc, bw_a) * jnp.minimum(thc, bh_a)
        union = twc * thc + bw_a * bh_a - inter
        rr = inter / jnp.maximum(union, 1e-12)
        upd = rr > best_r
        best_r = jnp.where(upd, rr, best_r)
        ba = jnp.where(upd, a, ba)
        bw_sel = jnp.where(upd, bw_a, bw_sel)
        bh_sel = jnp.where(upd, bh_a, bh_sel)
    tgt_w = jnp.log(jnp.maximum(twc, 1e-12) / bw_sel)
    tgt_h = jnp.log(jnp.maximum(thc, 1e-12) / bh_sel)

    m = jnp.where(valid, ba * _HW + cj * _W + ci, -1)
    mT = jnp.transpose(m, (1, 0))
    tcol = lax.broadcasted_iota(jnp.int32, (_T, 1), 0)
    tcolT = jnp.transpose(tcol, (1, 0))
    dup = jnp.max(((m == mT) & (tcolT > tcol)).astype(jnp.int32),
                  axis=1, keepdims=True)
    wins = dup == 0
    m_f = jnp.where(wins, m, -1).astype(f32)

    pad3 = jnp.full((_T, _NROW - 13), -1.0, dtype=f32)
    tab = jnp.concatenate(
        [m_f, tgt_x, tgt_y, tgt_w, tgt_h, wgt, tcls,
         tx, ty, tw, th, bw_sel, bh_sel, pad3], axis=1)
    tab = jnp.transpose(tab, (1, 0))
    tab = jnp.concatenate(
        [tab, jnp.full((_NROW, 2), -1.0, dtype=f32)], axis=1)
    tab_ref[0] = tab

    half_tw = tw * 0.5
    half_th = th * 0.5
    tl = jnp.where(valid, tx - half_tw, 1e30)
    tr = jnp.where(valid, tx + half_tw, -1e30)
    tt = ty - half_th
    tb = ty + half_th
    c1 = (tw * th) * (_POS_THRESH / (1.0 + _POS_THRESH))

    idx_row = lax.broadcasted_iota(jnp.int32, (1, _C), 1)
    gx = (idx_row % _W).astype(f32)
    gy = ((idx_row // _W) % _H).astype(f32)

    xy0 = xy_ref[0, 0:1, :]
    xy1 = xy_ref[0, 1:2, :]
    wh0 = wh_ref[0, 0:1, :]
    wh1 = wh_ref[0, 1:2, :]
    obj_r = obj_ref[0, 0:1, :]
    bw_row = brow_ref[0, 0:1, :]
    bh_row = brow_ref[0, 1:2, :]

    px = (gx + xy0) * (1.0 / _W)
    py = (gy + xy1) * (1.0 / _H)
    pw = jnp.exp(wh0) * bw_row
    ph = jnp.exp(wh1) * bh_row
    p_l = px - pw * 0.5
    p_r = px + pw * 0.5
    p_t = py - ph * 0.5
    p_b = py + ph * 0.5
    c0 = (pw * ph) * (_POS_THRESH / (1.0 + _POS_THRESH))

    l = jnp.maximum(p_l, tl)
    r = jnp.minimum(p_r, tr)
    t = jnp.maximum(p_t, tt)
    b = jnp.minimum(p_b, tb)
    inter = jnp.maximum(r - l, 0.0) * jnp.maximum(b - t, 0.0)
    over_f = (inter > (c0 + c1)).astype(f32)
    any_over = jnp.max(over_f, axis=0, keepdims=True) > 0.0
    tnoobj_ref[0, 0:1, :] = jnp.where(any_over, obj_r, 0.0)


def _sc_assign(B):
    f32 = jnp.float32
    mesh = plsc.VectorSubcoreMesh(core_axis_name="c", subcore_axis_name="s")
    n_tec = 32
    img_per_tec = B // n_tec
    out_t = tuple(jax.ShapeDtypeStruct(s, f32) for s in
                  [(2 * B, _C), (2 * B, _C), (B, _C), (B, _C), (B, _C),
                   (B, _C)])
    scr = ([pltpu.VMEM((_C,), f32) for _ in range(8)] +
           [pltpu.VMEM((_NROW, 2 * _T - 28), f32),
            pltpu.SemaphoreType.DMA])

    @functools.partial(
        pl.kernel, mesh=mesh, out_type=out_t, scratch_types=scr,
        compiler_params=pltpu.CompilerParams(needs_layout_passes=False))
    def k(xy2, wh2, obj1, noobj1, glc, tab3,
          o_xy, o_wh, o_g, o_obj, o_noobj, o_lab,
          s_x0, s_x1, s_w0, s_w1, s_obj, s_noobj, s_g, s_lab, s_tab, sem):
        wid = lax.axis_index("s") * 2 + lax.axis_index("c")
        for j in range(img_per_tec):
            bimg = wid * img_per_tec + j
            cps = [
                pltpu.make_async_copy(xy2.at[2 * bimg], s_x0, sem),
                pltpu.make_async_copy(xy2.at[2 * bimg + 1], s_x1, sem),
                pltpu.make_async_copy(wh2.at[2 * bimg], s_w0, sem),
                pltpu.make_async_copy(wh2.at[2 * bimg + 1], s_w1, sem),
                pltpu.make_async_copy(obj1.at[bimg], s_obj, sem),
                pltpu.make_async_copy(noobj1.at[bimg], s_noobj, sem),
                pltpu.make_async_copy(glc.at[0], s_g, sem),
                pltpu.make_async_copy(glc.at[1], s_lab, sem),
                pltpu.make_async_copy(tab3.at[bimg], s_tab, sem),
            ]
            for c in cps:
                c.start()
            for c in cps:
                c.wait()
            for v in range(2):
                sl = pl.ds(16 * v, 16)
                mi = s_tab[0, sl].astype(jnp.int32)
                mk = mi >= 0
                safe = jnp.maximum(mi, 0)
                tgx = s_tab[1, sl]
                tgy = s_tab[2, sl]
                tgw = s_tab[3, sl]
                tgh = s_tab[4, sl]
                wgt = s_tab[5, sl]
                tcl = s_tab[6, sl]
                txv = s_tab[7, sl]
                tyv = s_tab[8, sl]
                twv = s_tab[9, sl]
                thv = s_tab[10, sl]
                bwv = s_tab[11, sl]
                bhv = s_tab[12, sl]

                x0 = plsc.load_gather(s_x0, [safe])
                x1 = plsc.load_gather(s_x1, [safe])
                w0 = plsc.load_gather(s_w0, [safe])
                w1 = plsc.load_gather(s_w1, [safe])
                ov = plsc.load_gather(s_obj, [safe])

                gxv = (safe % 26).astype(f32)
                gyv = ((safe // 26) % 26).astype(f32)
                px = (gxv + x0) * (1.0 / _W)
                py = (gyv + x1) * (1.0 / _H)
                pw = jnp.exp(w0) * bwv * (1.0 / _W)
                ph = jnp.exp(w1) * bhv * (1.0 / _H)
                ll = jnp.maximum(px - pw * 0.5, txv - twv * 0.5)
                rr = jnp.minimum(px + pw * 0.5, txv + twv * 0.5)
                tt = jnp.maximum(py - ph * 0.5, tyv - thv * 0.5)
                bb = jnp.minimum(py + ph * 0.5, tyv + thv * 0.5)
                inter = (jnp.maximum(rr - ll, 0.0) *
                         jnp.maximum(bb - tt, 0.0))
                union = pw * ph + twv * thv - inter
                iou = jnp.where(union > 0,
                                inter / jnp.maximum(union, 1e-12), 0.0)

                plsc.store_scatter(s_x0, [safe], tgx, mask=mk)
                plsc.store_scatter(s_x1, [safe], tgy, mask=mk)
                plsc.store_scatter(s_w0, [safe], tgw, mask=mk)
                plsc.store_scatter(s_w1, [safe], tgh, mask=mk)
                plsc.store_scatter(s_g, [safe], wgt, mask=mk)
                plsc.store_scatter(s_obj, [safe], iou, mask=mk)
                plsc.store_scatter(s_noobj, [safe], ov, mask=mk)
                plsc.store_scatter(s_lab, [safe], tcl, mask=mk)

            ops = [
                pltpu.make_async_copy(s_x0, o_xy.at[2 * bimg], sem),
                pltpu.make_async_copy(s_x1, o_xy.at[2 * bimg + 1], sem),
                pltpu.make_async_copy(s_w0, o_wh.at[2 * bimg], sem),
                pltpu.make_async_copy(s_w1, o_wh.at[2 * bimg + 1], sem),
                pltpu.make_async_copy(s_g, o_g.at[bimg], sem),
                pltpu.make_async_copy(s_obj, o_obj.at[bimg], sem),
                pltpu.make_async_copy(s_noobj, o_noobj.at[bimg], sem),
                pltpu.make_async_copy(s_lab, o_lab.at[bimg], sem),
            ]
            for c in ops:
                c.start()
            for c in ops:
                c.wait()

    return k


def kernel(xy, wh, obj, truth, biases):
    B = xy.shape[0]
    f32 = jnp.float32
    xy_t = xy.reshape(B, _A, 2, _HW).transpose(0, 2, 1, 3).reshape(B, 2, _C)
    wh_t = wh.reshape(B, _A, 2, _HW).transpose(0, 2, 1, 3).reshape(B, 2, _C)
    obj_r = obj.reshape(B, 1, _C)
    bias_r = biases.reshape(1, 2 * _A)
    bi = biases.reshape(_A, 2)
    brow = jnp.stack([jnp.repeat(bi[:, 0] * (1.0 / _W), _HW),
                      jnp.repeat(bi[:, 1] * (1.0 / _H), _HW)], axis=0)
    brow = brow.reshape(1, 2, _C)

    tnoobj_base, table = pl.pallas_call(
        _tc_body,
        grid=(B,),
        in_specs=[
            pl.BlockSpec((1, 2, _C), lambda b: (b, 0, 0)),
            pl.BlockSpec((1, 2, _C), lambda b: (b, 0, 0)),
            pl.BlockSpec((1, 1, _C), lambda b: (b, 0, 0)),
            pl.BlockSpec((1, _T, 5), lambda b: (b, 0, 0)),
            pl.BlockSpec((1, 2 * _A), lambda b: (0, 0)),
            pl.BlockSpec((1, 2, _C), lambda b: (0, 0, 0)),
        ],
        out_specs=(
            pl.BlockSpec((1, 1, _C), lambda b: (b, 0, 0)),
            pl.BlockSpec((1, _NROW, 2 * _T - 28), lambda b: (b, 0, 0)),
        ),
        out_shape=(
            jax.ShapeDtypeStruct((B, 1, _C), f32),
            jax.ShapeDtypeStruct((B, _NROW, 2 * _T - 28), f32),
        ),
    )(xy_t, wh_t, obj_r, truth, bias_r, brow)

    glc = jnp.stack([jnp.zeros((_C,), f32), jnp.full((_C,), -1.0, f32)])
    xyo, who, g, tobj, tnoobj, tlabel = _sc_assign(B)(
        xy_t.reshape(2 * B, _C), wh_t.reshape(2 * B, _C),
        obj_r.reshape(B, _C), tnoobj_base.reshape(B, _C), glc, table)

    t_xy = xyo.reshape(B, 2, _A, _HW).transpose(0, 2, 1, 3).reshape(
        B, 2 * _A, _H, _W)
    t_wh = who.reshape(B, 2, _A, _HW).transpose(0, 2, 1, 3).reshape(
        B, 2 * _A, _H, _W)
    g4 = g.reshape(B, 1, _A, _HW)
    t_w = jnp.broadcast_to(g4, (B, 2, _A, _HW)).transpose(0, 2, 1, 3).reshape(
        B, 2 * _A, _H, _W)
    return (
        t_xy,
        t_wh,
        t_w,
        tobj.reshape(B, _A, _H, _W),
        tnoobj.reshape(B, _A, _H, _W),
        tlabel.reshape(B, _A, _H, _W),
    )
